# baseline (speedup 1.0000x reference)
import dataclasses
import functools

import jax
import jax.numpy as jnp
from jax import lax
from jax.experimental import pallas as pl
from jax.experimental.pallas import tpu as pltpu
from jax.experimental.pallas import tpu_sc as plsc

N = 10000
E = 320000
D = 128

NC = 2
NS = 16
NW = NC * NS
EP = 327680
SBE = 320
SBS = 8
SLEN = 40
NSB = EP // (NW * SBE)
EPW = EP // NW
IROWS = EP // SLEN
NPAD = 10240
RPT = NPAD // NS
ZROWS = 128

def _sc_compiler_params():
    cp = pltpu.CompilerParams()
    if "needs_layout_passes" in pltpu.CompilerParams.__dataclass_fields__:
        cp = dataclasses.replace(cp, needs_layout_passes=False)
    return cp


GSB = 400
GNS = E // NW // GSB


def _sc_gather_body(src1_hbm, w_hbm, cj_hbm, g1_hbm, g2_hbm,
                    idxa_v, idxb_v, rowsa_v, rowsb_v, g2ba_v, g2bb_v, cj_v,
                    si, sg, sw, sw2):
    idx_v = [idxa_v, idxb_v]
    rows_v = [rowsa_v, rowsb_v]
    g2b_v = [g2ba_v, g2bb_v]
    wid = lax.axis_index("s") * NC + lax.axis_index("c")
    edge_base = wid * (E // NW)
    pltpu.sync_copy(cj_hbm, cj_v)

    ic = [None, None]
    wb1 = [None, None]
    wb2 = [None, None]
    ic[0] = pltpu.async_copy(src1_hbm.at[pl.ds(edge_base, GSB)], idx_v[0],
                             si.at[0])
    for sb in range(GNS):
        s = sb % 2
        n = (sb + 1) % 2
        if sb + 1 < GNS:
            e = edge_base + (sb + 1) * GSB
            ic[n] = pltpu.async_copy(src1_hbm.at[pl.ds(e, GSB)], idx_v[n],
                                     si.at[n])
        ic[s].wait()
        if wb1[s] is not None:
            wb1[s].wait()
            wb2[s].wait()
        gathers = [
            pltpu.async_copy(w_hbm.at[idx_v[s].at[pl.ds(k * 80, 80)]],
                             rows_v[s].at[pl.ds(k * 80, 80)], sg.at[s])
            for k in range(GSB // 80)
        ]
        i1 = idx_v[s]
        gb = g2b_v[s]

        @pl.loop(0, GSB // 16)
        def _(j):
            iv = i1[pl.ds(j * 16, 16)]
            gb[pl.ds(j * 16, 16)] = plsc.load_gather(cj_v, [iv])
        for g in gathers:
            g.wait()
        e = edge_base + sb * GSB
        wb1[s] = pltpu.async_copy(rows_v[s], g1_hbm.at[pl.ds(e, GSB)], sw.at[s])
        wb2[s] = pltpu.async_copy(g2b_v[s], g2_hbm.at[pl.ds(e, GSB)], sw2.at[s])
    for s in range(2):
        if wb1[s] is not None:
            wb1[s].wait()
            wb2[s].wait()


@functools.cache
def _build_sc_gather():
    mesh = plsc.VectorSubcoreMesh(
        core_axis_name="c", subcore_axis_name="s",
        num_cores=NC, num_subcores=NS)
    return pl.kernel(
        _sc_gather_body,
        out_type=[jax.ShapeDtypeStruct((E, D), jnp.float32),
                  jax.ShapeDtypeStruct((E,), jnp.float32)],
        mesh=mesh,
        scratch_types=[pltpu.VMEM((GSB,), jnp.int32),
                       pltpu.VMEM((GSB,), jnp.int32),
                       pltpu.VMEM((GSB, D), jnp.float32),
                       pltpu.VMEM((GSB, D), jnp.float32),
                       pltpu.VMEM((GSB,), jnp.float32),
                       pltpu.VMEM((GSB,), jnp.float32),
                       pltpu.VMEM((N,), jnp.float32),
                       pltpu.SemaphoreType.DMA((2,)),
                       pltpu.SemaphoreType.DMA((2,)),
                       pltpu.SemaphoreType.DMA((2,)),
                       pltpu.SemaphoreType.DMA((2,))],
        compiler_params=_sc_compiler_params(),
    )


BE = 512
NBE = E // BE


def _tc_pre_body(x_ref, attn_ref, wTb_ref, pwrw_ref, mp_ref, s1_ref):
    x = x_ref[...]
    y = jnp.dot(x, pwrw_ref[...], preferred_element_type=jnp.float32)
    sg = 1.0 / (1.0 + jnp.exp(-y))
    pa = sg[:, 0:1]
    ra = sg[:, 1:2]
    rf = jnp.dot(x.astype(jnp.bfloat16), wTb_ref[...],
                 preferred_element_type=jnp.float32)
    mp_ref[...] = rf * (ra * attn_ref[...])
    s1_ref[...] = pa


_tc_pre = pl.pallas_call(
    _tc_pre_body,
    grid=(NBE,),
    in_specs=[
        pl.BlockSpec((BE, D), lambda i: (i, 0)),
        pl.BlockSpec((BE, 1), lambda i: (i, 0)),
        pl.BlockSpec((D, D), lambda i: (0, 0)),
        pl.BlockSpec((D, 2), lambda i: (0, 0)),
    ],
    out_specs=[
        pl.BlockSpec((BE, D), lambda i: (i, 0)),
        pl.BlockSpec((BE, 1), lambda i: (i, 0)),
    ],
    out_shape=[jax.ShapeDtypeStruct((EP, D), jnp.float32),
               jax.ShapeDtypeStruct((E, 1), jnp.float32)],
)


def _tc_mix_body(g1_ref, mp_ref, s1_ref, g2_ref, m_ref):
    m_ref[...] = (g1_ref[...] * s1_ref[...] + mp_ref[...]) * g2_ref[...]


_tc_mix = pl.pallas_call(
    _tc_mix_body,
    grid=(NBE,),
    in_specs=[
        pl.BlockSpec((BE, D), lambda i: (i, 0)),
        pl.BlockSpec((BE, D), lambda i: (i, 0)),
        pl.BlockSpec((BE, 1), lambda i: (i, 0)),
        pl.BlockSpec((BE, 1), lambda i: (i, 0)),
    ],
    out_specs=pl.BlockSpec((BE, D), lambda i: (i, 0)),
    out_shape=jax.ShapeDtypeStruct((EP, D), jnp.float32),
)


def _sc_scatter_body(dst2_hbm, m_hbm, z_hbm, out_hbm,
                     idxa_v, idxb_v, rowsa_v, rowsb_v, acc_sh,
                     sia, sib, sra, srb, ssa, ssb):
    cid = lax.axis_index("c")
    sid = lax.axis_index("s")
    wid = sid * NC + cid
    row_base = wid * (EPW // SLEN)
    edge_base = wid * EPW

    @pl.loop(0, RPT // ZROWS)
    def _(k):
        pltpu.sync_copy(z_hbm, acc_sh.at[pl.ds(sid * RPT + k * ZROWS, ZROWS)])
    plsc.subcore_barrier()

    @pl.loop(0, NSB)
    def _(sb):
        r = row_base + sb * SBS
        e = edge_base + sb * SBE
        ia = pltpu.async_copy(dst2_hbm.at[pl.ds(r, SBS)], idxa_v, sia)
        ca = pltpu.async_copy(m_hbm.at[pl.ds(e, SBE)], rowsa_v, sra)
        ia.wait()
        ca.wait()
        sca = [pltpu.async_copy(rowsa_v.at[pl.ds(k * SLEN, SLEN)],
                                acc_sh.at[idxa_v.at[k]], ssa, add=True)
               for k in range(SBS)]
        for c in sca:
            c.wait()

    plsc.subcore_barrier()

    @pl.loop(0, RPT // ZROWS)
    def _(k):
        r = sid * RPT + k * ZROWS
        pltpu.sync_copy(acc_sh.at[pl.ds(r, ZROWS)], out_hbm.at[cid].at[pl.ds(r, ZROWS)])


@functools.cache
def _build_sc_scatter():
    mesh = plsc.VectorSubcoreMesh(
        core_axis_name="c", subcore_axis_name="s",
        num_cores=NC, num_subcores=NS)
    return pl.kernel(
        _sc_scatter_body,
        out_type=jax.ShapeDtypeStruct((NC, NPAD, D), jnp.float32),
        mesh=mesh,
        scratch_types=[pltpu.VMEM((SBS, SLEN), jnp.int32),
                       pltpu.VMEM((SBS, SLEN), jnp.int32),
                       pltpu.VMEM((SBE, D), jnp.float32),
                       pltpu.VMEM((SBE, D), jnp.float32),
                       pltpu.VMEM_SHARED((NPAD, D), jnp.float32),
                       pltpu.SemaphoreType.DMA,
                       pltpu.SemaphoreType.DMA,
                       pltpu.SemaphoreType.DMA,
                       pltpu.SemaphoreType.DMA,
                       pltpu.SemaphoreType.DMA,
                       pltpu.SemaphoreType.DMA],
    )


BN = 1000
NBN = N // BN


def _tc_combine_body(p_ref, ci_ref, o_ref):
    o_ref[...] = (p_ref[0] + p_ref[1]) * ci_ref[...]


_tc_combine = pl.pallas_call(
    _tc_combine_body,
    grid=(NBN,),
    in_specs=[
        pl.BlockSpec((NC, BN, D), lambda i: (0, i, 0)),
        pl.BlockSpec((BN, 1), lambda i: (i, 0)),
    ],
    out_specs=pl.BlockSpec((BN, D), lambda i: (i, 0)),
    out_shape=jax.ShapeDtypeStruct((N, D), jnp.float32),
)


def kernel(edge_index, attn, review_feat, cj, ci, weight, prob_score_w,
           review_score_w, review_w):
    dst = jnp.concatenate(
        [edge_index[1], N + (jnp.arange(EP - E, dtype=jnp.int32) % (NPAD - N))])
    attn2 = attn.reshape(E, 1)
    zeros = jnp.zeros((ZROWS, D), jnp.float32)

    wTb = review_w.T.astype(jnp.bfloat16)
    pwrw = jnp.concatenate([prob_score_w.T, review_score_w.T], axis=1)

    mp, s1 = _tc_pre(review_feat, attn2, wTb, pwrw)
    g1, g2 = _build_sc_gather()(edge_index[0], weight, cj.reshape(N))
    m = _tc_mix(g1, mp, s1, g2.reshape(E, 1))
    parts = _build_sc_scatter()(dst.reshape(IROWS, SLEN), m, zeros)
    return _tc_combine(parts, ci)


# speedup vs baseline: 1.4290x; 1.4290x over previous
import dataclasses
import functools

import jax
import jax.numpy as jnp
from jax import lax
from jax.experimental import pallas as pl
from jax.experimental.pallas import tpu as pltpu
from jax.experimental.pallas import tpu_sc as plsc

N = 10000
E = 320000
D = 128

NC = 2
NS = 16
NW = NC * NS
EP = 327680
SBE = 320
SBS = 8
SLEN = 40
NSB = EP // (NW * SBE)
EPW = EP // NW
IROWS = EP // SLEN
NPAD = 10240
RPT = NPAD // NS
ZROWS = 128

def _sc_compiler_params():
    cp = pltpu.CompilerParams()
    if "needs_layout_passes" in pltpu.CompilerParams.__dataclass_fields__:
        cp = dataclasses.replace(cp, needs_layout_passes=False)
    return cp


GSB = 400
GNS = E // NW // GSB


def _sc_gather_body(src1_hbm, w_hbm, cj_hbm, g1_hbm, g2_hbm,
                    idxa_v, idxb_v, rowsa_v, rowsb_v, g2ba_v, g2bb_v, cj_v,
                    si, sg, sw, sw2):
    idx_v = [idxa_v, idxb_v]
    rows_v = [rowsa_v, rowsb_v]
    g2b_v = [g2ba_v, g2bb_v]
    wid = lax.axis_index("s") * NC + lax.axis_index("c")
    edge_base = wid * (E // NW)
    pltpu.sync_copy(cj_hbm, cj_v)

    ic = [None, None]
    wb1 = [None, None]
    wb2 = [None, None]
    ic[0] = pltpu.async_copy(src1_hbm.at[pl.ds(edge_base, GSB)], idx_v[0],
                             si.at[0])
    for sb in range(GNS):
        s = sb % 2
        n = (sb + 1) % 2
        if sb + 1 < GNS:
            e = edge_base + (sb + 1) * GSB
            ic[n] = pltpu.async_copy(src1_hbm.at[pl.ds(e, GSB)], idx_v[n],
                                     si.at[n])
        ic[s].wait()
        if wb1[s] is not None:
            wb1[s].wait()
            wb2[s].wait()
        gathers = [
            pltpu.async_copy(w_hbm.at[idx_v[s].at[pl.ds(k * 80, 80)]],
                             rows_v[s].at[pl.ds(k * 80, 80)], sg.at[s])
            for k in range(GSB // 80)
        ]
        i1 = idx_v[s]
        gb = g2b_v[s]

        @pl.loop(0, GSB // 16)
        def _(j):
            iv = i1[pl.ds(j * 16, 16)]
            gb[pl.ds(j * 16, 16)] = plsc.load_gather(cj_v, [iv])
        for g in gathers:
            g.wait()
        e = edge_base + sb * GSB
        wb1[s] = pltpu.async_copy(rows_v[s], g1_hbm.at[pl.ds(e, GSB)], sw.at[s])
        wb2[s] = pltpu.async_copy(g2b_v[s], g2_hbm.at[pl.ds(e, GSB)], sw2.at[s])
    for s in range(2):
        if wb1[s] is not None:
            wb1[s].wait()
            wb2[s].wait()


@functools.cache
def _build_sc_gather():
    mesh = plsc.VectorSubcoreMesh(
        core_axis_name="c", subcore_axis_name="s",
        num_cores=NC, num_subcores=NS)
    return pl.kernel(
        _sc_gather_body,
        out_type=[jax.ShapeDtypeStruct((E, D), jnp.float32),
                  jax.ShapeDtypeStruct((E,), jnp.float32)],
        mesh=mesh,
        scratch_types=[pltpu.VMEM((GSB,), jnp.int32),
                       pltpu.VMEM((GSB,), jnp.int32),
                       pltpu.VMEM((GSB, D), jnp.float32),
                       pltpu.VMEM((GSB, D), jnp.float32),
                       pltpu.VMEM((GSB,), jnp.float32),
                       pltpu.VMEM((GSB,), jnp.float32),
                       pltpu.VMEM((N,), jnp.float32),
                       pltpu.SemaphoreType.DMA((2,)),
                       pltpu.SemaphoreType.DMA((2,)),
                       pltpu.SemaphoreType.DMA((2,)),
                       pltpu.SemaphoreType.DMA((2,))],
        compiler_params=_sc_compiler_params(),
    )


BE = 512
NBE = E // BE


def _tc_main_body(x_ref, attn_ref, g1_ref, g2_ref, wTb_ref, pw_ref, rw_ref,
                  m_ref):
    x = x_ref[...]
    pa_lin = jnp.dot(x, pw_ref[...], preferred_element_type=jnp.float32)
    ra_lin = jnp.dot(x, rw_ref[...], preferred_element_type=jnp.float32)
    pa = 1.0 / (1.0 + jnp.exp(-pa_lin))
    ra = 1.0 / (1.0 + jnp.exp(-ra_lin))
    rf = jnp.dot(x.astype(jnp.bfloat16), wTb_ref[...],
                 preferred_element_type=jnp.float32)
    m_ref[...] = (g1_ref[...] * pa + rf * (ra * attn_ref[...])) * g2_ref[...]


_tc_main = pl.pallas_call(
    _tc_main_body,
    grid=(NBE,),
    in_specs=[
        pl.BlockSpec((BE, D), lambda i: (i, 0)),
        pl.BlockSpec((BE, 1), lambda i: (i, 0)),
        pl.BlockSpec((BE, D), lambda i: (i, 0)),
        pl.BlockSpec((BE, 1), lambda i: (i, 0)),
        pl.BlockSpec((D, D), lambda i: (0, 0)),
        pl.BlockSpec((D, 1), lambda i: (0, 0)),
        pl.BlockSpec((D, 1), lambda i: (0, 0)),
    ],
    out_specs=pl.BlockSpec((BE, D), lambda i: (i, 0)),
    out_shape=jax.ShapeDtypeStruct((EP, D), jnp.float32),
)


def _sc_scatter_body(dst2_hbm, m_hbm, z_hbm, out_hbm,
                     idxa_v, rowsa_v, acc_sh, sia, sra, ssa):
    cid = lax.axis_index("c")
    sid = lax.axis_index("s")
    wid = sid * NC + cid
    row_base = wid * (EPW // SLEN)
    edge_base = wid * EPW

    @pl.loop(0, RPT // ZROWS)
    def _(k):
        pltpu.sync_copy(z_hbm, acc_sh.at[pl.ds(sid * RPT + k * ZROWS, ZROWS)])
    plsc.subcore_barrier()

    @pl.loop(0, NSB)
    def _(sb):
        r = row_base + sb * SBS
        e = edge_base + sb * SBE
        ia = pltpu.async_copy(dst2_hbm.at[pl.ds(r, SBS)], idxa_v, sia)
        ca = pltpu.async_copy(m_hbm.at[pl.ds(e, SBE)], rowsa_v, sra)
        ia.wait()
        ca.wait()
        sca = [pltpu.async_copy(rowsa_v.at[pl.ds(k * SLEN, SLEN)],
                                acc_sh.at[idxa_v.at[k]], ssa, add=True)
               for k in range(SBS)]
        for c in sca:
            c.wait()

    plsc.subcore_barrier()

    @pl.loop(0, RPT // ZROWS)
    def _(k):
        r = sid * RPT + k * ZROWS
        pltpu.sync_copy(acc_sh.at[pl.ds(r, ZROWS)], out_hbm.at[cid].at[pl.ds(r, ZROWS)])


@functools.cache
def _build_sc_scatter():
    mesh = plsc.VectorSubcoreMesh(
        core_axis_name="c", subcore_axis_name="s",
        num_cores=NC, num_subcores=NS)
    return pl.kernel(
        _sc_scatter_body,
        out_type=jax.ShapeDtypeStruct((NC, NPAD, D), jnp.float32),
        mesh=mesh,
        scratch_types=[pltpu.VMEM((SBS, SLEN), jnp.int32),
                       pltpu.VMEM((SBE, D), jnp.float32),
                       pltpu.VMEM_SHARED((NPAD, D), jnp.float32),
                       pltpu.SemaphoreType.DMA,
                       pltpu.SemaphoreType.DMA,
                       pltpu.SemaphoreType.DMA],
    )


BN = 1000
NBN = N // BN


def _tc_combine_body(p_ref, ci_ref, o_ref):
    o_ref[...] = (p_ref[0] + p_ref[1]) * ci_ref[...]


_tc_combine = pl.pallas_call(
    _tc_combine_body,
    grid=(NBN,),
    in_specs=[
        pl.BlockSpec((NC, BN, D), lambda i: (0, i, 0)),
        pl.BlockSpec((BN, 1), lambda i: (i, 0)),
    ],
    out_specs=pl.BlockSpec((BN, D), lambda i: (i, 0)),
    out_shape=jax.ShapeDtypeStruct((N, D), jnp.float32),
)


def kernel(edge_index, attn, review_feat, cj, ci, weight, prob_score_w,
           review_score_w, review_w):
    attn2 = attn.reshape(E, 1)
    zeros = jnp.zeros((ZROWS, D), jnp.float32)
    wTb = review_w.T.astype(jnp.bfloat16)

    dst = jnp.concatenate(
        [edge_index[1], N + (jnp.arange(EP - E, dtype=jnp.int32) % (NPAD - N))])

    g1, g2 = _build_sc_gather()(edge_index[0], weight, cj.reshape(N))
    m = _tc_main(review_feat, attn2, g1, g2.reshape(E, 1), wTb,
                 prob_score_w.T, review_score_w.T)
    parts = _build_sc_scatter()(dst.reshape(IROWS, SLEN), m, zeros)
    return _tc_combine(parts, ci)


# speedup vs baseline: 1.6778x; 1.1741x over previous
import dataclasses
import functools

import jax
import jax.numpy as jnp
from jax import lax
from jax.experimental import pallas as pl
from jax.experimental.pallas import tpu as pltpu
from jax.experimental.pallas import tpu_sc as plsc

N = 10000
E = 320000
D = 128

NC = 2
NS = 16
NW = NC * NS
SBE = 320
SBS = 8
SLEN = 40
TSB = E // SBE
IROWS = E // SLEN
NPAD = 10240
RPT = NPAD // NS
ZROWS = 128

def _sc_compiler_params():
    cp = pltpu.CompilerParams()
    if "needs_layout_passes" in pltpu.CompilerParams.__dataclass_fields__:
        cp = dataclasses.replace(cp, needs_layout_passes=False)
    return cp


GSB = 400
GNS = E // NW // GSB


def _sc_gather_body(src1_hbm, w_hbm, cj_hbm, g1_hbm, g2_hbm,
                    idxa_v, idxb_v, rowsa_v, rowsb_v, g2ba_v, g2bb_v, cj_v,
                    si, sg, sw, sw2):
    idx_v = [idxa_v, idxb_v]
    rows_v = [rowsa_v, rowsb_v]
    g2b_v = [g2ba_v, g2bb_v]
    wid = lax.axis_index("s") * NC + lax.axis_index("c")
    edge_base = wid * (E // NW)
    pltpu.sync_copy(cj_hbm, cj_v)

    ic = [None, None]
    wb1 = [None, None]
    wb2 = [None, None]
    ic[0] = pltpu.async_copy(src1_hbm.at[pl.ds(edge_base, GSB)], idx_v[0],
                             si.at[0])
    for sb in range(GNS):
        s = sb % 2
        n = (sb + 1) % 2
        if sb + 1 < GNS:
            e = edge_base + (sb + 1) * GSB
            ic[n] = pltpu.async_copy(src1_hbm.at[pl.ds(e, GSB)], idx_v[n],
                                     si.at[n])
        ic[s].wait()
        if wb1[s] is not None:
            wb1[s].wait()
            wb2[s].wait()
        gathers = [
            pltpu.async_copy(w_hbm.at[idx_v[s].at[pl.ds(k * 80, 80)]],
                             rows_v[s].at[pl.ds(k * 80, 80)], sg.at[s])
            for k in range(GSB // 80)
        ]
        i1 = idx_v[s]
        gb = g2b_v[s]

        @pl.loop(0, GSB // 16)
        def _(j):
            iv = i1[pl.ds(j * 16, 16)]
            gb[pl.ds(j * 16, 16)] = plsc.load_gather(cj_v, [iv])
        for g in gathers:
            g.wait()
        e = edge_base + sb * GSB
        wb1[s] = pltpu.async_copy(rows_v[s], g1_hbm.at[pl.ds(e, GSB)], sw.at[s])
        wb2[s] = pltpu.async_copy(g2b_v[s], g2_hbm.at[pl.ds(e, GSB)], sw2.at[s])
    for s in range(2):
        if wb1[s] is not None:
            wb1[s].wait()
            wb2[s].wait()


@functools.cache
def _build_sc_gather():
    mesh = plsc.VectorSubcoreMesh(
        core_axis_name="c", subcore_axis_name="s",
        num_cores=NC, num_subcores=NS)
    return pl.kernel(
        _sc_gather_body,
        out_type=[jax.ShapeDtypeStruct((E, D), jnp.float32),
                  jax.ShapeDtypeStruct((E,), jnp.float32)],
        mesh=mesh,
        scratch_types=[pltpu.VMEM((GSB,), jnp.int32),
                       pltpu.VMEM((GSB,), jnp.int32),
                       pltpu.VMEM((GSB, D), jnp.float32),
                       pltpu.VMEM((GSB, D), jnp.float32),
                       pltpu.VMEM((GSB,), jnp.float32),
                       pltpu.VMEM((GSB,), jnp.float32),
                       pltpu.VMEM((N,), jnp.float32),
                       pltpu.SemaphoreType.DMA((2,)),
                       pltpu.SemaphoreType.DMA((2,)),
                       pltpu.SemaphoreType.DMA((2,)),
                       pltpu.SemaphoreType.DMA((2,))],
        compiler_params=_sc_compiler_params(),
    )


BE = 1000
NBE = E // BE


def _tc_main_body(x_ref, attn_ref, g1_ref, g2_ref, wTb_ref, pw_ref, rw_ref,
                  m_ref):
    x = x_ref[...]
    pa_lin = jnp.dot(x, pw_ref[...], preferred_element_type=jnp.float32)
    ra_lin = jnp.dot(x, rw_ref[...], preferred_element_type=jnp.float32)
    pa = 1.0 / (1.0 + jnp.exp(-pa_lin))
    ra = 1.0 / (1.0 + jnp.exp(-ra_lin))
    rf = jnp.dot(x.astype(jnp.bfloat16), wTb_ref[...],
                 preferred_element_type=jnp.float32)
    m_ref[...] = (g1_ref[...] * pa + rf * (ra * attn_ref[...])) * g2_ref[...]


_tc_main = pl.pallas_call(
    _tc_main_body,
    grid=(NBE,),
    in_specs=[
        pl.BlockSpec((BE, D), lambda i: (i, 0)),
        pl.BlockSpec((BE, 1), lambda i: (i, 0)),
        pl.BlockSpec((BE, D), lambda i: (i, 0)),
        pl.BlockSpec((BE, 1), lambda i: (i, 0)),
        pl.BlockSpec((D, D), lambda i: (0, 0)),
        pl.BlockSpec((D, 1), lambda i: (0, 0)),
        pl.BlockSpec((D, 1), lambda i: (0, 0)),
    ],
    out_specs=pl.BlockSpec((BE, D), lambda i: (i, 0)),
    out_shape=jax.ShapeDtypeStruct((E, D), jnp.float32),
)


def _sc_scatter_body(dst2_hbm, m_hbm, z_hbm, out_hbm,
                     idxa_v, rowsa_v, acc_sh, sia, sra, ssa):
    cid = lax.axis_index("c")
    sid = lax.axis_index("s")
    wid = sid * NC + cid

    @pl.loop(0, RPT // ZROWS)
    def _(k):
        pltpu.sync_copy(z_hbm, acc_sh.at[pl.ds(sid * RPT + k * ZROWS, ZROWS)])
    plsc.subcore_barrier()

    nsb = 31 + jnp.where(wid < TSB - 31 * NW, 1, 0)

    @pl.loop(0, nsb)
    def _(sb):
        j = wid + NW * sb
        r = j * SBS
        e = j * SBE
        ia = pltpu.async_copy(dst2_hbm.at[pl.ds(r, SBS)], idxa_v, sia)
        ca = pltpu.async_copy(m_hbm.at[pl.ds(e, SBE)], rowsa_v, sra)
        ia.wait()
        ca.wait()
        sca = [pltpu.async_copy(rowsa_v.at[pl.ds(k * SLEN, SLEN)],
                                acc_sh.at[idxa_v.at[k]], ssa, add=True)
               for k in range(SBS)]
        for c in sca:
            c.wait()

    plsc.subcore_barrier()

    @pl.loop(0, RPT // ZROWS)
    def _(k):
        r = sid * RPT + k * ZROWS
        pltpu.sync_copy(acc_sh.at[pl.ds(r, ZROWS)], out_hbm.at[cid].at[pl.ds(r, ZROWS)])


@functools.cache
def _build_sc_scatter():
    mesh = plsc.VectorSubcoreMesh(
        core_axis_name="c", subcore_axis_name="s",
        num_cores=NC, num_subcores=NS)
    return pl.kernel(
        _sc_scatter_body,
        out_type=jax.ShapeDtypeStruct((NC, NPAD, D), jnp.float32),
        mesh=mesh,
        scratch_types=[pltpu.VMEM((SBS, SLEN), jnp.int32),
                       pltpu.VMEM((SBE, D), jnp.float32),
                       pltpu.VMEM_SHARED((NPAD, D), jnp.float32),
                       pltpu.SemaphoreType.DMA,
                       pltpu.SemaphoreType.DMA,
                       pltpu.SemaphoreType.DMA],
    )


BN = 1000
NBN = N // BN


def _tc_combine_body(p_ref, ci_ref, o_ref):
    o_ref[...] = (p_ref[0] + p_ref[1]) * ci_ref[...]


_tc_combine = pl.pallas_call(
    _tc_combine_body,
    grid=(NBN,),
    in_specs=[
        pl.BlockSpec((NC, BN, D), lambda i: (0, i, 0)),
        pl.BlockSpec((BN, 1), lambda i: (i, 0)),
    ],
    out_specs=pl.BlockSpec((BN, D), lambda i: (i, 0)),
    out_shape=jax.ShapeDtypeStruct((N, D), jnp.float32),
)


def kernel(edge_index, attn, review_feat, cj, ci, weight, prob_score_w,
           review_score_w, review_w):
    attn2 = attn.reshape(E, 1)
    zeros = jnp.zeros((ZROWS, D), jnp.float32)
    wTb = review_w.T.astype(jnp.bfloat16)

    g1, g2 = _build_sc_gather()(edge_index[0], weight, cj.reshape(N))
    m = _tc_main(review_feat, attn2, g1, g2.reshape(E, 1), wTb,
                 prob_score_w.T, review_score_w.T)
    parts = _build_sc_scatter()(edge_index[1].reshape(IROWS, SLEN), m, zeros)
    return _tc_combine(parts, ci)


# speedup vs baseline: 1.8464x; 1.1005x over previous
import dataclasses
import functools

import jax
import jax.numpy as jnp
from jax import lax
from jax.experimental import pallas as pl
from jax.experimental.pallas import tpu as pltpu
from jax.experimental.pallas import tpu_sc as plsc

N = 10000
E = 320000
D = 128

NC = 2
NS = 16
NW = NC * NS
SBE = 320
SBS = 8
SLEN = 40
TSB = E // SBE
IROWS = E // SLEN
NPAD = 10240
RPT = NPAD // NS
ZROWS = 128

def _sc_compiler_params():
    cp = pltpu.CompilerParams()
    if "needs_layout_passes" in pltpu.CompilerParams.__dataclass_fields__:
        cp = dataclasses.replace(cp, needs_layout_passes=False)
    return cp


GSB = 400
GNS = E // NW // GSB


def _sc_gather_body(src1_hbm, w_hbm, cj_hbm, g1_hbm, g2_hbm,
                    idxa_v, idxb_v, rowsa_v, rowsb_v, g2ba_v, g2bb_v, cj_v,
                    si, sg, sw, sw2):
    idx_v = [idxa_v, idxb_v]
    rows_v = [rowsa_v, rowsb_v]
    g2b_v = [g2ba_v, g2bb_v]
    wid = lax.axis_index("s") * NC + lax.axis_index("c")
    edge_base = wid * (E // NW)
    pltpu.sync_copy(cj_hbm, cj_v)

    ic = [None, None]
    wb1 = [None, None]
    wb2 = [None, None]
    ic[0] = pltpu.async_copy(src1_hbm.at[pl.ds(edge_base, GSB)], idx_v[0],
                             si.at[0])
    for sb in range(GNS):
        s = sb % 2
        n = (sb + 1) % 2
        if sb + 1 < GNS:
            e = edge_base + (sb + 1) * GSB
            ic[n] = pltpu.async_copy(src1_hbm.at[pl.ds(e, GSB)], idx_v[n],
                                     si.at[n])
        ic[s].wait()
        if wb1[s] is not None:
            wb1[s].wait()
            wb2[s].wait()
        gathers = [
            pltpu.async_copy(w_hbm.at[idx_v[s].at[pl.ds(k * 80, 80)]],
                             rows_v[s].at[pl.ds(k * 80, 80)], sg.at[s])
            for k in range(GSB // 80)
        ]
        i1 = idx_v[s]
        gb = g2b_v[s]

        @pl.loop(0, GSB // 16)
        def _(j):
            iv = i1[pl.ds(j * 16, 16)]
            gb[pl.ds(j * 16, 16)] = plsc.load_gather(cj_v, [iv])
        for g in gathers:
            g.wait()
        e = edge_base + sb * GSB
        wb1[s] = pltpu.async_copy(rows_v[s], g1_hbm.at[pl.ds(e, GSB)], sw.at[s])
        wb2[s] = pltpu.async_copy(g2b_v[s], g2_hbm.at[pl.ds(e, GSB)], sw2.at[s])
    for s in range(2):
        if wb1[s] is not None:
            wb1[s].wait()
            wb2[s].wait()


@functools.cache
def _build_sc_gather():
    mesh = plsc.VectorSubcoreMesh(
        core_axis_name="c", subcore_axis_name="s",
        num_cores=NC, num_subcores=NS)
    return pl.kernel(
        _sc_gather_body,
        out_type=[jax.ShapeDtypeStruct((E, D), jnp.float32),
                  jax.ShapeDtypeStruct((E,), jnp.float32)],
        mesh=mesh,
        scratch_types=[pltpu.VMEM((GSB,), jnp.int32),
                       pltpu.VMEM((GSB,), jnp.int32),
                       pltpu.VMEM((GSB, D), jnp.float32),
                       pltpu.VMEM((GSB, D), jnp.float32),
                       pltpu.VMEM((GSB,), jnp.float32),
                       pltpu.VMEM((GSB,), jnp.float32),
                       pltpu.VMEM((N,), jnp.float32),
                       pltpu.SemaphoreType.DMA((2,)),
                       pltpu.SemaphoreType.DMA((2,)),
                       pltpu.SemaphoreType.DMA((2,)),
                       pltpu.SemaphoreType.DMA((2,))],
        compiler_params=_sc_compiler_params(),
    )


BE = 1600
NBE = E // BE


def _tc_main_body(x_ref, attn_ref, g1_ref, g2_ref, wTb_ref, pw_ref, rw_ref,
                  m_ref):
    x = x_ref[...]
    pa_lin = jnp.dot(x, pw_ref[...], preferred_element_type=jnp.float32)
    ra_lin = jnp.dot(x, rw_ref[...], preferred_element_type=jnp.float32)
    pa = 1.0 / (1.0 + jnp.exp(-pa_lin))
    ra = 1.0 / (1.0 + jnp.exp(-ra_lin))
    rf = jnp.dot(x.astype(jnp.bfloat16), wTb_ref[...],
                 preferred_element_type=jnp.float32)
    m_ref[...] = (g1_ref[...] * pa + rf * (ra * attn_ref[...])) * g2_ref[...]


_tc_main = pl.pallas_call(
    _tc_main_body,
    grid=(NBE,),
    in_specs=[
        pl.BlockSpec((BE, D), lambda i: (i, 0)),
        pl.BlockSpec((BE, 1), lambda i: (i, 0)),
        pl.BlockSpec((BE, D), lambda i: (i, 0)),
        pl.BlockSpec((BE, 1), lambda i: (i, 0)),
        pl.BlockSpec((D, D), lambda i: (0, 0)),
        pl.BlockSpec((D, 1), lambda i: (0, 0)),
        pl.BlockSpec((D, 1), lambda i: (0, 0)),
    ],
    out_specs=pl.BlockSpec((BE, D), lambda i: (i, 0)),
    out_shape=jax.ShapeDtypeStruct((E, D), jnp.float32),
)


def _sc_scatter_body(dst2_hbm, m_hbm, z_hbm, out_hbm,
                     idxa_v, rowsa_v, acc_sh, sia, sra, ssa):
    cid = lax.axis_index("c")
    sid = lax.axis_index("s")
    wid = sid * NC + cid

    @pl.loop(0, RPT // ZROWS)
    def _(k):
        pltpu.sync_copy(z_hbm, acc_sh.at[pl.ds(sid * RPT + k * ZROWS, ZROWS)])
    plsc.subcore_barrier()

    nsb = 31 + jnp.where(wid < TSB - 31 * NW, 1, 0)

    @pl.loop(0, nsb)
    def _(sb):
        j = wid + NW * sb
        r = j * SBS
        e = j * SBE
        ia = pltpu.async_copy(dst2_hbm.at[pl.ds(r, SBS)], idxa_v, sia)
        ca = pltpu.async_copy(m_hbm.at[pl.ds(e, SBE)], rowsa_v, sra)
        ia.wait()
        ca.wait()
        sca = [pltpu.async_copy(rowsa_v.at[pl.ds(k * SLEN, SLEN)],
                                acc_sh.at[idxa_v.at[k]], ssa, add=True)
               for k in range(SBS)]
        for c in sca:
            c.wait()

    plsc.subcore_barrier()

    @pl.loop(0, RPT // ZROWS)
    def _(k):
        r = sid * RPT + k * ZROWS
        pltpu.sync_copy(acc_sh.at[pl.ds(r, ZROWS)], out_hbm.at[cid].at[pl.ds(r, ZROWS)])


@functools.cache
def _build_sc_scatter():
    mesh = plsc.VectorSubcoreMesh(
        core_axis_name="c", subcore_axis_name="s",
        num_cores=NC, num_subcores=NS)
    return pl.kernel(
        _sc_scatter_body,
        out_type=jax.ShapeDtypeStruct((NC, NPAD, D), jnp.float32),
        mesh=mesh,
        scratch_types=[pltpu.VMEM((SBS, SLEN), jnp.int32),
                       pltpu.VMEM((SBE, D), jnp.float32),
                       pltpu.VMEM_SHARED((NPAD, D), jnp.float32),
                       pltpu.SemaphoreType.DMA,
                       pltpu.SemaphoreType.DMA,
                       pltpu.SemaphoreType.DMA],
    )


BN = 1000
NBN = N // BN


def _tc_combine_body(p_ref, ci_ref, o_ref):
    o_ref[...] = (p_ref[0] + p_ref[1]) * ci_ref[...]


_tc_combine = pl.pallas_call(
    _tc_combine_body,
    grid=(NBN,),
    in_specs=[
        pl.BlockSpec((NC, BN, D), lambda i: (0, i, 0)),
        pl.BlockSpec((BN, 1), lambda i: (i, 0)),
    ],
    out_specs=pl.BlockSpec((BN, D), lambda i: (i, 0)),
    out_shape=jax.ShapeDtypeStruct((N, D), jnp.float32),
)


def kernel(edge_index, attn, review_feat, cj, ci, weight, prob_score_w,
           review_score_w, review_w):
    attn2 = attn.reshape(E, 1)
    zeros = jnp.zeros((ZROWS, D), jnp.float32)
    wTb = review_w.T.astype(jnp.bfloat16)

    g1, g2 = _build_sc_gather()(edge_index[0], weight, cj.reshape(N))
    m = _tc_main(review_feat, attn2, g1, g2.reshape(E, 1), wTb,
                 prob_score_w.T, review_score_w.T)
    parts = _build_sc_scatter()(edge_index[1].reshape(IROWS, SLEN), m, zeros)
    return _tc_combine(parts, ci)
